# Initial kernel scaffold; baseline (speedup 1.0000x reference)
import jax
import jax.numpy as jnp
from jax.experimental import pallas as pl


def kernel(x, edge_index, edge_type, W_sl, b_sl, W1, b1, gamma, beta, W2, b2):
    raise NotImplementedError("write your pallas kernel here")


# speedup vs baseline: 2.1456x; 2.1456x over previous
import jax
import jax.numpy as jnp
from jax import lax
from jax.experimental import pallas as pl
from jax.experimental.pallas import tpu as pltpu
from jax.experimental.pallas import tpu_sc as plsc

_N = 10000
_E = 320000
_D = 128
_R = 4
_BN_EPS = 1e-5

_NQ = 4
_QR = _N // _NQ
_ROWS = _R * _QR
_ROWS_PAD = 10240
_DUMMY = _ROWS
_NTILES = 16
_RPT = _ROWS_PAD // _NTILES
_CH = 128
_EPT = 20480
_E_PAD = _EPT * _NTILES
_NCHUNKS = _EPT // _CH
_ZROWS = 128


def _sc_agg_kernel(src_hbm, dst_hbm, typ_hbm, x_hbm, out_hbm,
                   srcv, dstv, typv, slotv, rows, zbuf, agg_sh, sem):
    c = lax.axis_index("c")
    s = lax.axis_index("s")
    s_base = s * _EPT

    def _zbody(i, carry):
        for l in range(_D // 16):
            zbuf[i, pl.ds(l * 16, 16)] = jnp.zeros((16,), jnp.float32)
        return carry
    lax.fori_loop(0, _ZROWS, _zbody, 0)

    for p in range(2):
        q = c * 2 + p
        base_row = q * _QR

        for b in range(_RPT // _ZROWS):
            pltpu.sync_copy(zbuf,
                            agg_sh.at[pl.ds(s * _RPT + b * _ZROWS, _ZROWS)])
        plsc.subcore_barrier()

        def _chunk(j, carry):
            off = s_base + j * _CH
            pltpu.sync_copy(src_hbm.at[pl.ds(off, _CH)], srcv)
            pltpu.sync_copy(dst_hbm.at[pl.ds(off, _CH)], dstv)
            pltpu.sync_copy(typ_hbm.at[pl.ds(off, _CH)], typv)
            for k in range(_CH // 16):
                d = dstv[pl.ds(k * 16, 16)]
                t = typv[pl.ds(k * 16, 16)]
                in_r = (d >= base_row) & (d < base_row + _QR)
                slot = jnp.where(in_r, t * _QR + (d - base_row), _DUMMY)
                slotv[pl.ds(k * 16, 16)] = slot
            pltpu.async_copy(x_hbm.at[srcv], rows, sem).wait()
            pltpu.sync_copy(rows, agg_sh.at[slotv], add=True)
            return carry
        lax.fori_loop(0, _NCHUNKS, _chunk, 0)
        plsc.subcore_barrier()

        pltpu.sync_copy(agg_sh.at[pl.ds(s * _RPT, _RPT)],
                        out_hbm.at[q, pl.ds(s * _RPT, _RPT)])


def _run_sc_agg(src, dst, typ, x):
    mesh = plsc.VectorSubcoreMesh(core_axis_name="c", subcore_axis_name="s",
                                  num_cores=2)
    fn = pl.kernel(
        _sc_agg_kernel,
        mesh=mesh,
        out_type=jax.ShapeDtypeStruct((_NQ, _ROWS_PAD, _D), jnp.float32),
        scratch_types=[
            pltpu.VMEM((_CH,), jnp.int32),
            pltpu.VMEM((_CH,), jnp.int32),
            pltpu.VMEM((_CH,), jnp.int32),
            pltpu.VMEM((_CH,), jnp.int32),
            pltpu.VMEM((_CH, _D), jnp.float32),
            pltpu.VMEM((_ZROWS, _D), jnp.float32),
            pltpu.VMEM_SHARED((_ROWS_PAD, _D), jnp.float32),
            pltpu.SemaphoreType.DMA,
        ],
    )
    return fn(src, dst, typ, x)


def _tc_body(x_ref, a_ref, wsl_ref, bsl_ref, w1_ref, b1_ref, g_ref, be_ref,
             w2_ref, b2_ref, o_ref):
    x = x_ref[...]
    acc = jnp.dot(x, wsl_ref[...],
                  preferred_element_type=jnp.float32) + bsl_ref[...][None, :]
    for r in range(_R):
        agg = jnp.concatenate(
            [a_ref[q, r * _QR:(r + 1) * _QR, :] for q in range(_NQ)], axis=0)
        h = x + agg
        h = jnp.dot(h, w1_ref[r],
                    preferred_element_type=jnp.float32) + b1_ref[r][None, :]
        mean = jnp.mean(h, axis=0)
        hc = h - mean[None, :]
        var = jnp.mean(hc * hc, axis=0)
        inv = lax.rsqrt(var + _BN_EPS)
        h = hc * (inv * g_ref[r])[None, :] + be_ref[r][None, :]
        h = jnp.maximum(h, 0.0)
        acc = acc + jnp.dot(h, w2_ref[r],
                            preferred_element_type=jnp.float32) + b2_ref[r][None, :]
    o_ref[...] = acc


def _tc_mlp(x, agg, W_sl, b_sl, W1, b1, gamma, beta, W2, b2):
    return pl.pallas_call(
        _tc_body,
        out_shape=jax.ShapeDtypeStruct((_N, _D), jnp.float32),
    )(x, agg, W_sl, b_sl, W1, b1, gamma, beta, W2, b2)


def kernel(x, edge_index, edge_type, W_sl, b_sl, W1, b1, gamma, beta, W2, b2):
    src = edge_index[0]
    dst = edge_index[1]
    pad = _E_PAD - _E
    src_p = jnp.concatenate([src, jnp.zeros((pad,), jnp.int32)])
    dst_p = jnp.concatenate([dst, jnp.full((pad,), _N, jnp.int32)])
    typ_p = jnp.concatenate([edge_type, jnp.zeros((pad,), jnp.int32)])
    agg = _run_sc_agg(src_p, dst_p, typ_p, x)
    return _tc_mlp(x, agg, W_sl, b_sl, W1, b1, gamma, beta, W2, b2)


# speedup vs baseline: 2.4449x; 1.1395x over previous
import jax
import jax.numpy as jnp
from jax import lax
from jax.experimental import pallas as pl
from jax.experimental.pallas import tpu as pltpu
from jax.experimental.pallas import tpu_sc as plsc

_N = 10000
_E = 320000
_D = 128
_R = 4
_BN_EPS = 1e-5

_NQ = 4
_QR = _N // _NQ
_ROWS = _R * _QR
_ROWS_PAD = 10240
_DUMMY = _ROWS
_NTILES = 16
_RPT = _ROWS_PAD // _NTILES
_CH = 128
_EPT = 20480
_E_PAD = _EPT * _NTILES
_NCH = _EPT // _CH
_ZROWS = 64


def _sc_agg_kernel(src_hbm, dst_hbm, typ_hbm, x_hbm, out_hbm,
                   srcv0, srcv1, dstv0, dstv1, typv0, typv1,
                   slotv0, slotv1, rows0, rows1, zbuf, agg_sh,
                   gsem0, gsem1):
    c = lax.axis_index("c")
    s = lax.axis_index("s")
    row_base = s * _NCH
    srcv = (srcv0, srcv1)
    dstv = (dstv0, dstv1)
    typv = (typv0, typv1)
    slotv = (slotv0, slotv1)
    rows = (rows0, rows1)
    gsem = (gsem0, gsem1)

    def _zbody(i, carry):
        for l in range(_D // 16):
            zbuf[i, pl.ds(l * 16, 16)] = jnp.zeros((16,), jnp.float32)
        return carry
    lax.fori_loop(0, _ZROWS, _zbody, 0)

    for p in range(2):
        q = c * 2 + p
        base_row = q * _QR

        for b in range(_RPT // _ZROWS):
            zoff = pl.multiple_of(s * _RPT + b * _ZROWS, _ZROWS)
            pltpu.sync_copy(zbuf, agg_sh.at[pl.ds(zoff, _ZROWS)])
        plsc.subcore_barrier()

        def _stage(i, sj):
            roff = row_base + sj
            pltpu.sync_copy(src_hbm.at[roff], srcv[i])
            pltpu.sync_copy(dst_hbm.at[roff], dstv[i])
            pltpu.sync_copy(typ_hbm.at[roff], typv[i])
            iota16 = lax.iota(jnp.int32, 16)
            for k in range(_CH // 16):
                d = dstv[i][pl.ds(k * 16, 16)]
                t = typv[i][pl.ds(k * 16, 16)]
                in_r = (d >= base_row) & (d < base_row + _QR)
                dummy = _DUMMY + (k % 15) * 16 + iota16
                slot = jnp.where(in_r, t * _QR + (d - base_row), dummy)
                slotv[i][pl.ds(k * 16, 16)] = slot
            pltpu.async_copy(x_hbm.at[srcv[i]], rows[i], gsem[i])

        def _drain_scatter(i):
            pltpu.make_async_copy(x_hbm.at[srcv[i]], rows[i], gsem[i]).wait()
            pltpu.sync_copy(rows[i], agg_sh.at[slotv[i]], add=True)

        _stage(0, 0)

        def _pair(j, carry):
            for u in range(2):
                sj = 2 * j + u

                @pl.when(sj + 1 < _NCH)
                def _():
                    _stage(u ^ 1, sj + 1)
                _drain_scatter(u)
            return carry
        lax.fori_loop(0, _NCH // 2, _pair, 0)
        plsc.subcore_barrier()

        woff = pl.multiple_of(s * _RPT, _RPT)
        pltpu.sync_copy(agg_sh.at[pl.ds(woff, _RPT)],
                        out_hbm.at[q, pl.ds(woff, _RPT)])


def _run_sc_agg(src2d, dst2d, typ2d, x):
    mesh = plsc.VectorSubcoreMesh(core_axis_name="c", subcore_axis_name="s",
                                  num_cores=2)
    fn = pl.kernel(
        _sc_agg_kernel,
        mesh=mesh,
        out_type=jax.ShapeDtypeStruct((_NQ, _ROWS_PAD, _D), jnp.float32),
        scratch_types=[
            pltpu.VMEM((_CH,), jnp.int32),
            pltpu.VMEM((_CH,), jnp.int32),
            pltpu.VMEM((_CH,), jnp.int32),
            pltpu.VMEM((_CH,), jnp.int32),
            pltpu.VMEM((_CH,), jnp.int32),
            pltpu.VMEM((_CH,), jnp.int32),
            pltpu.VMEM((_CH,), jnp.int32),
            pltpu.VMEM((_CH,), jnp.int32),
            pltpu.VMEM((_CH, _D), jnp.float32),
            pltpu.VMEM((_CH, _D), jnp.float32),
            pltpu.VMEM((_ZROWS, _D), jnp.float32),
            pltpu.VMEM_SHARED((_ROWS_PAD, _D), jnp.float32),
            pltpu.SemaphoreType.DMA,
            pltpu.SemaphoreType.DMA,
        ],
    )
    return fn(src2d, dst2d, typ2d, x)


def _tc_body(x_ref, a_ref, wsl_ref, bsl_ref, w1_ref, b1_ref, g_ref, be_ref,
             w2_ref, b2_ref, o_ref):
    x = x_ref[...]
    acc = jnp.dot(x, wsl_ref[...],
                  preferred_element_type=jnp.float32) + bsl_ref[...][None, :]
    for r in range(_R):
        agg = jnp.concatenate(
            [a_ref[q, r * _QR:(r + 1) * _QR, :] for q in range(_NQ)], axis=0)
        h = x + agg
        h = jnp.dot(h, w1_ref[r],
                    preferred_element_type=jnp.float32) + b1_ref[r][None, :]
        mean = jnp.mean(h, axis=0)
        hc = h - mean[None, :]
        var = jnp.mean(hc * hc, axis=0)
        inv = lax.rsqrt(var + _BN_EPS)
        h = hc * (inv * g_ref[r])[None, :] + be_ref[r][None, :]
        h = jnp.maximum(h, 0.0)
        acc = acc + jnp.dot(h, w2_ref[r],
                            preferred_element_type=jnp.float32) + b2_ref[r][None, :]
    o_ref[...] = acc


def _tc_mlp(x, agg, W_sl, b_sl, W1, b1, gamma, beta, W2, b2):
    return pl.pallas_call(
        _tc_body,
        out_shape=jax.ShapeDtypeStruct((_N, _D), jnp.float32),
    )(x, agg, W_sl, b_sl, W1, b1, gamma, beta, W2, b2)


def kernel(x, edge_index, edge_type, W_sl, b_sl, W1, b1, gamma, beta, W2, b2):
    src = edge_index[0]
    dst = edge_index[1]
    pad = _E_PAD - _E
    src_p = jnp.concatenate([src, jnp.zeros((pad,), jnp.int32)])
    dst_p = jnp.concatenate([dst, jnp.full((pad,), _N, jnp.int32)])
    typ_p = jnp.concatenate([edge_type, jnp.zeros((pad,), jnp.int32)])
    src2d = src_p.reshape(_E_PAD // _CH, _CH)
    dst2d = dst_p.reshape(_E_PAD // _CH, _CH)
    typ2d = typ_p.reshape(_E_PAD // _CH, _CH)
    agg = _run_sc_agg(src2d, dst2d, typ2d, x)
    return _tc_mlp(x, agg, W_sl, b_sl, W1, b1, gamma, beta, W2, b2)


# speedup vs baseline: 2.7025x; 1.1054x over previous
import jax
import jax.numpy as jnp
from jax import lax
from jax.experimental import pallas as pl
from jax.experimental.pallas import tpu as pltpu
from jax.experimental.pallas import tpu_sc as plsc

_N = 10000
_E = 320000
_D = 128
_R = 4
_BN_EPS = 1e-5

_NQ = 4
_QR = _N // _NQ
_ROWS = _R * _QR
_ROWS_PAD = 10240
_DUMMY = _ROWS
_NTILES = 16
_RPT = _ROWS_PAD // _NTILES
_CH = 128
_EPT = 20480
_E_PAD = _EPT * _NTILES
_NCH = _EPT // _CH
_GRP = 8
_NGRP = _NCH // _GRP
_ZROWS = 32


def _sc_agg_kernel(src_hbm, dst_hbm, typ_hbm, x_hbm, out_hbm,
                   srcg0, srcg1, dstg0, dstg1, typg0, typg1,
                   slotg0, slotg1, rows0, rows1, zbuf, agg_sh,
                   gsem0, gsem1, ssem0, ssem1):
    c = lax.axis_index("c")
    s = lax.axis_index("s")
    row_base = s * _NCH
    srcg = (srcg0, srcg1)
    dstg = (dstg0, dstg1)
    typg = (typg0, typg1)
    slotg = (slotg0, slotg1)
    rows = (rows0, rows1)
    gsem = (gsem0, gsem1)
    ssem = (ssem0, ssem1)

    def _zbody(i, carry):
        for l in range(_D // 16):
            zbuf[i, pl.ds(l * 16, 16)] = jnp.zeros((16,), jnp.float32)
        return carry
    lax.fori_loop(0, _ZROWS, _zbody, 0)

    def _fire_gather(u, t):
        pltpu.async_copy(x_hbm.at[srcg[u].at[t]], rows[t % 2], gsem[t % 2])

    def _wait_gather(u, t):
        pltpu.make_async_copy(x_hbm.at[srcg[u].at[t]], rows[t % 2],
                              gsem[t % 2]).wait()

    def _fire_scatter(u, t):
        pltpu.async_copy(rows[t % 2], agg_sh.at[slotg[u].at[t]],
                         ssem[t % 2], add=True)

    def _wait_scatter(u, t):
        pltpu.make_async_copy(rows[t % 2], agg_sh.at[slotg[u].at[t]],
                              ssem[t % 2]).wait()

    for p in range(2):
        q = c * 2 + p
        base_row = q * _QR

        for b in range(_RPT // _ZROWS):
            zoff = pl.multiple_of(s * _RPT + b * _ZROWS, _ZROWS)
            pltpu.sync_copy(zbuf, agg_sh.at[pl.ds(zoff, _ZROWS)])
        plsc.subcore_barrier()

        iota16 = lax.iota(jnp.int32, 16)

        def _slots(u, t):
            for k in range(_CH // 16):
                d = dstg[u][t, pl.ds(k * 16, 16)]
                tt = typg[u][t, pl.ds(k * 16, 16)]
                in_r = (d >= base_row) & (d < base_row + _QR)
                dummy = _DUMMY + (k % 15) * 16 + iota16
                slot = jnp.where(in_r, tt * _QR + (d - base_row), dummy)
                slotg[u][t, pl.ds(k * 16, 16)] = slot

        def _group(u, g):
            grow = row_base + g * _GRP
            pltpu.sync_copy(src_hbm.at[pl.ds(grow, _GRP)], srcg[u])
            pltpu.sync_copy(dst_hbm.at[pl.ds(grow, _GRP)], dstg[u])
            pltpu.sync_copy(typ_hbm.at[pl.ds(grow, _GRP)], typg[u])
            for t in range(_GRP):
                if t >= 2:
                    _wait_scatter(u, t - 2)
                else:
                    @pl.when(g > 0)
                    def _():
                        _wait_scatter(u ^ 1, _GRP - 2 + t)
                _slots(u, t)
                _fire_gather(u, t)
                if t >= 1:
                    _wait_gather(u, t - 1)
                    _fire_scatter(u, t - 1)
                else:
                    @pl.when(g > 0)
                    def _():
                        _wait_gather(u ^ 1, _GRP - 1)
                        _fire_scatter(u ^ 1, _GRP - 1)

        def _gpair(j, carry):
            _group(0, 2 * j)
            _group(1, 2 * j + 1)
            return carry
        lax.fori_loop(0, _NGRP // 2, _gpair, 0)
        _wait_gather(1, _GRP - 1)
        _fire_scatter(1, _GRP - 1)
        _wait_scatter(1, _GRP - 2)
        _wait_scatter(1, _GRP - 1)
        plsc.subcore_barrier()

        woff = pl.multiple_of(s * _RPT, _RPT)
        pltpu.sync_copy(agg_sh.at[pl.ds(woff, _RPT)],
                        out_hbm.at[q, pl.ds(woff, _RPT)])


def _run_sc_agg(src2d, dst2d, typ2d, x):
    mesh = plsc.VectorSubcoreMesh(core_axis_name="c", subcore_axis_name="s",
                                  num_cores=2)
    fn = pl.kernel(
        _sc_agg_kernel,
        mesh=mesh,
        out_type=jax.ShapeDtypeStruct((_NQ, _ROWS_PAD, _D), jnp.float32),
        scratch_types=[
            pltpu.VMEM((_GRP, _CH), jnp.int32),
            pltpu.VMEM((_GRP, _CH), jnp.int32),
            pltpu.VMEM((_GRP, _CH), jnp.int32),
            pltpu.VMEM((_GRP, _CH), jnp.int32),
            pltpu.VMEM((_GRP, _CH), jnp.int32),
            pltpu.VMEM((_GRP, _CH), jnp.int32),
            pltpu.VMEM((_GRP, _CH), jnp.int32),
            pltpu.VMEM((_GRP, _CH), jnp.int32),
            pltpu.VMEM((_CH, _D), jnp.float32),
            pltpu.VMEM((_CH, _D), jnp.float32),
            pltpu.VMEM((_ZROWS, _D), jnp.float32),
            pltpu.VMEM_SHARED((_ROWS_PAD, _D), jnp.float32),
            pltpu.SemaphoreType.DMA,
            pltpu.SemaphoreType.DMA,
            pltpu.SemaphoreType.DMA,
            pltpu.SemaphoreType.DMA,
        ],
    )
    return fn(src2d, dst2d, typ2d, x)


def _tc_body(x_ref, a_ref, wsl_ref, bsl_ref, w1_ref, b1_ref, g_ref, be_ref,
             w2_ref, b2_ref, o_ref):
    x = x_ref[...]
    acc = jnp.dot(x, wsl_ref[...],
                  preferred_element_type=jnp.float32) + bsl_ref[...][None, :]
    for r in range(_R):
        agg = jnp.concatenate(
            [a_ref[q, r * _QR:(r + 1) * _QR, :] for q in range(_NQ)], axis=0)
        h = x + agg
        h = jnp.dot(h, w1_ref[r],
                    preferred_element_type=jnp.float32) + b1_ref[r][None, :]
        mean = jnp.mean(h, axis=0)
        hc = h - mean[None, :]
        var = jnp.mean(hc * hc, axis=0)
        inv = lax.rsqrt(var + _BN_EPS)
        h = hc * (inv * g_ref[r])[None, :] + be_ref[r][None, :]
        h = jnp.maximum(h, 0.0)
        acc = acc + jnp.dot(h, w2_ref[r],
                            preferred_element_type=jnp.float32) + b2_ref[r][None, :]
    o_ref[...] = acc


def _tc_mlp(x, agg, W_sl, b_sl, W1, b1, gamma, beta, W2, b2):
    return pl.pallas_call(
        _tc_body,
        out_shape=jax.ShapeDtypeStruct((_N, _D), jnp.float32),
    )(x, agg, W_sl, b_sl, W1, b1, gamma, beta, W2, b2)


def kernel(x, edge_index, edge_type, W_sl, b_sl, W1, b1, gamma, beta, W2, b2):
    src = edge_index[0]
    dst = edge_index[1]
    pad = _E_PAD - _E
    src_p = jnp.concatenate([src, jnp.zeros((pad,), jnp.int32)])
    dst_p = jnp.concatenate([dst, jnp.full((pad,), _N, jnp.int32)])
    typ_p = jnp.concatenate([edge_type, jnp.zeros((pad,), jnp.int32)])
    src2d = src_p.reshape(_E_PAD // _CH, _CH)
    dst2d = dst_p.reshape(_E_PAD // _CH, _CH)
    typ2d = typ_p.reshape(_E_PAD // _CH, _CH)
    agg = _run_sc_agg(src2d, dst2d, typ2d, x)
    return _tc_mlp(x, agg, W_sl, b_sl, W1, b1, gamma, beta, W2, b2)
